# Initial kernel scaffold; baseline (speedup 1.0000x reference)
import jax
import jax.numpy as jnp
from jax.experimental import pallas as pl


def kernel(x1, x2, x3, output1_w, output1_scale, output1_bias, output2_w, output2_scale, output2_bias, output3_w, output3_scale, output3_bias, merge1_w, merge1_scale, merge1_bias, merge2_w, merge2_scale, merge2_bias):
    raise NotImplementedError("write your pallas kernel here")


# speedup vs baseline: 1.1909x; 1.1909x over previous
import functools

import jax
import jax.numpy as jnp
from jax.experimental import pallas as pl
from jax.experimental.pallas import tpu as pltpu


_SLOPE = 0.1


def _neck_kernel(x1_ref, x2_ref, x3_ref,
                 w1_ref, s1_ref, b1_ref,
                 w2_ref, s2_ref, b2_ref,
                 w3_ref, s3_ref, b3_ref,
                 wm1_ref, sm1_ref, bm1_ref,
                 wm2_ref, sm2_ref, bm2_ref,
                 u21_ref, u32_ref,
                 mneg1_ref, mpos1_ref, mneg2_ref, mpos2_ref,
                 o1_ref, o2_ref, o3_ref,
                 xpad1_ref, xpad2_ref,
                 *, H1, W1, H2, W2, pad1, pad2):
    HW1 = H1 * W1
    HW2 = H2 * W2

    def bn_lrelu(y, s_ref, b_ref):
        y = y * s_ref[...] + b_ref[...]
        return jnp.where(y >= 0.0, y, _SLOPE * y)

    x1 = x1_ref[0].astype(jnp.bfloat16)
    x2 = x2_ref[0].astype(jnp.bfloat16)
    x3 = x3_ref[0].astype(jnp.bfloat16)
    o1 = bn_lrelu(jnp.dot(w1_ref[...], x1, preferred_element_type=jnp.float32),
                  s1_ref, b1_ref)
    o2 = bn_lrelu(jnp.dot(w2_ref[...], x2, preferred_element_type=jnp.float32),
                  s2_ref, b2_ref)
    o3 = bn_lrelu(jnp.dot(w3_ref[...], x3, preferred_element_type=jnp.float32),
                  s3_ref, b3_ref)
    o3_ref[0] = o3

    def merge(o_fine, coarse_bf, u_ref, xpad_ref, w_ref, s_ref, b_ref,
              pad, W, HW, mneg_ref, mpos_ref):
        up = jnp.dot(coarse_bf, u_ref[...], preferred_element_type=jnp.float32)
        m = (o_fine + up).astype(jnp.bfloat16)

        xpad_ref[...] = jnp.zeros_like(xpad_ref)
        xpad_ref[:, pad:pad + HW] = m

        mneg = mneg_ref[...]
        mpos = mpos_ref[...]
        patches = []
        for dh in (-1, 0, 1):
            for dw in (-1, 0, 1):
                off = dh * W + dw
                shifted = xpad_ref[:, pad + off:pad + off + HW]
                if dw == -1:
                    shifted = shifted * mneg
                elif dw == 1:
                    shifted = shifted * mpos
                patches.append(shifted)
        p = jnp.concatenate(patches, axis=0)
        y = jnp.dot(w_ref[...], p, preferred_element_type=jnp.float32)
        return bn_lrelu(y, s_ref, b_ref)

    o2m = merge(o2, o3.astype(jnp.bfloat16), u32_ref, xpad2_ref,
                wm2_ref, sm2_ref, bm2_ref, pad2, W2, HW2, mneg2_ref, mpos2_ref)
    o2_ref[0] = o2m
    o1m = merge(o1, o2m.astype(jnp.bfloat16), u21_ref, xpad1_ref,
                wm1_ref, sm1_ref, bm1_ref, pad1, W1, HW1, mneg1_ref, mpos1_ref)
    o1_ref[0] = o1m


def _upsample_matrix(hc, wc, h, w):
    hi = (jnp.arange(h) * hc) // h
    wi = (jnp.arange(w) * wc) // w
    src = (hi[:, None] * wc + wi[None, :]).reshape(-1)
    return (jnp.arange(hc * wc)[:, None] == src[None, :]).astype(jnp.bfloat16)


def kernel(x1, x2, x3,
           output1_w, output1_scale, output1_bias,
           output2_w, output2_scale, output2_bias,
           output3_w, output3_scale, output3_bias,
           merge1_w, merge1_scale, merge1_bias,
           merge2_w, merge2_scale, merge2_bias):
    N, C1, H1, W1 = x1.shape
    _, C2, H2, W2 = x2.shape
    _, C3, H3, W3 = x3.shape
    HW1, HW2, HW3 = H1 * W1, H2 * W2, H3 * W3
    Cout = output1_w.shape[1]

    x1f = x1.reshape(N, C1, HW1)
    x2f = x2.reshape(N, C2, HW2)
    x3f = x3.reshape(N, C3, HW3)

    def pack1x1(w):
        return w.T.astype(jnp.bfloat16)

    def pack3x3(w):
        cout = w.shape[3]
        return (jnp.transpose(w, (3, 0, 1, 2)).reshape(cout, -1)
                .astype(jnp.bfloat16))

    w1 = pack1x1(output1_w)
    w2 = pack1x1(output2_w)
    w3 = pack1x1(output3_w)
    wm1 = pack3x3(merge1_w)
    wm2 = pack3x3(merge2_w)

    u21 = _upsample_matrix(H2, W2, H1, W1)
    u32 = _upsample_matrix(H3, W3, H2, W2)

    def col_masks(h, w):
        ww = jnp.arange(h * w) % w
        mneg = (ww > 0).astype(jnp.bfloat16).reshape(1, -1)
        mpos = (ww < w - 1).astype(jnp.bfloat16).reshape(1, -1)
        return mneg, mpos

    mneg1, mpos1 = col_masks(H1, W1)
    mneg2, mpos2 = col_masks(H2, W2)

    pad1 = max(128, W1 + 1)
    pad2 = max(128, W2 + 1)

    const = lambda *shape: pl.BlockSpec(shape, lambda n: tuple(0 for _ in shape))
    pern = lambda c, hw: pl.BlockSpec((1, c, hw), lambda n: (n, 0, 0))

    o1f, o2f, o3f = pl.pallas_call(
        functools.partial(_neck_kernel, H1=H1, W1=W1, H2=H2, W2=W2,
                          pad1=pad1, pad2=pad2),
        out_shape=[jax.ShapeDtypeStruct((N, Cout, HW1), jnp.float32),
                   jax.ShapeDtypeStruct((N, Cout, HW2), jnp.float32),
                   jax.ShapeDtypeStruct((N, Cout, HW3), jnp.float32)],
        grid=(N,),
        in_specs=[
            pern(C1, HW1), pern(C2, HW2), pern(C3, HW3),
            const(Cout, C1), const(Cout, 1), const(Cout, 1),
            const(Cout, C2), const(Cout, 1), const(Cout, 1),
            const(Cout, C3), const(Cout, 1), const(Cout, 1),
            const(Cout, 9 * Cout), const(Cout, 1), const(Cout, 1),
            const(Cout, 9 * Cout), const(Cout, 1), const(Cout, 1),
            const(HW2, HW1), const(HW3, HW2),
            const(1, HW1), const(1, HW1), const(1, HW2), const(1, HW2),
        ],
        out_specs=[pern(Cout, HW1), pern(Cout, HW2), pern(Cout, HW3)],
        scratch_shapes=[
            pltpu.VMEM((Cout, HW1 + 2 * pad1), jnp.bfloat16),
            pltpu.VMEM((Cout, HW2 + 2 * pad2), jnp.bfloat16),
        ],
        compiler_params=pltpu.CompilerParams(
            dimension_semantics=("parallel",),
            vmem_limit_bytes=100 * 1024 * 1024,
        ),
    )(x1f, x2f, x3f,
      w1, output1_scale.reshape(-1, 1), output1_bias.reshape(-1, 1),
      w2, output2_scale.reshape(-1, 1), output2_bias.reshape(-1, 1),
      w3, output3_scale.reshape(-1, 1), output3_bias.reshape(-1, 1),
      wm1, merge1_scale.reshape(-1, 1), merge1_bias.reshape(-1, 1),
      wm2, merge2_scale.reshape(-1, 1), merge2_bias.reshape(-1, 1),
      u21, u32, mneg1, mpos1, mneg2, mpos2)

    return [o1f.reshape(N, Cout, H1, W1),
            o2f.reshape(N, Cout, H2, W2),
            o3f.reshape(N, Cout, H3, W3)]


# speedup vs baseline: 1.3671x; 1.1480x over previous
import functools

import jax
import jax.numpy as jnp
import numpy as np
from jax.experimental import pallas as pl
from jax.experimental.pallas import tpu as pltpu


_SLOPE = 0.1


def _neck_kernel(x1_ref, x2_ref, x3_ref,
                 w1_ref, s1_ref, b1_ref,
                 w2_ref, s2_ref, b2_ref,
                 w3_ref, s3_ref, b3_ref,
                 wm1_ref, sm1_ref, bm1_ref,
                 wm2_ref, sm2_ref, bm2_ref,
                 u21_ref, u32_ref,
                 mneg1_ref, mpos1_ref, mneg2_ref, mpos2_ref,
                 o1_ref, o2_ref, o3_ref,
                 xpad1_ref, xpad2_ref,
                 *, H1, W1, H2, W2, pad1, pad2):
    HW1 = H1 * W1
    HW2 = H2 * W2

    def bn_lrelu(y, s_ref, b_ref):
        y = y * s_ref[...] + b_ref[...]
        return jnp.where(y >= 0.0, y, _SLOPE * y)

    x1 = x1_ref[0].astype(jnp.bfloat16)
    x2 = x2_ref[0].astype(jnp.bfloat16)
    x3 = x3_ref[0].astype(jnp.bfloat16)
    o1 = bn_lrelu(jnp.dot(w1_ref[...], x1, preferred_element_type=jnp.float32),
                  s1_ref, b1_ref)
    o2 = bn_lrelu(jnp.dot(w2_ref[...], x2, preferred_element_type=jnp.float32),
                  s2_ref, b2_ref)
    o3 = bn_lrelu(jnp.dot(w3_ref[...], x3, preferred_element_type=jnp.float32),
                  s3_ref, b3_ref)
    o3_ref[0] = o3

    def merge(o_fine, coarse_bf, u_ref, xpad_ref, w_ref, s_ref, b_ref,
              pad, W, HW, mneg_ref, mpos_ref):
        up = jnp.dot(coarse_bf, u_ref[...], preferred_element_type=jnp.float32)
        m = (o_fine + up).astype(jnp.bfloat16)

        xpad_ref[...] = jnp.zeros_like(xpad_ref)
        xpad_ref[:, pad:pad + HW] = m

        mneg = mneg_ref[...]
        mpos = mpos_ref[...]
        patches = []
        for dh in (-1, 0, 1):
            for dw in (-1, 0, 1):
                off = dh * W + dw
                shifted = xpad_ref[:, pad + off:pad + off + HW]
                if dw == -1:
                    shifted = shifted * mneg
                elif dw == 1:
                    shifted = shifted * mpos
                patches.append(shifted)
        p = jnp.concatenate(patches, axis=0)
        y = jnp.dot(w_ref[...], p, preferred_element_type=jnp.float32)
        return bn_lrelu(y, s_ref, b_ref)

    o2m = merge(o2, o3.astype(jnp.bfloat16), u32_ref, xpad2_ref,
                wm2_ref, sm2_ref, bm2_ref, pad2, W2, HW2, mneg2_ref, mpos2_ref)
    o2_ref[0] = o2m
    o1m = merge(o1, o2m.astype(jnp.bfloat16), u21_ref, xpad1_ref,
                wm1_ref, sm1_ref, bm1_ref, pad1, W1, HW1, mneg1_ref, mpos1_ref)
    o1_ref[0] = o1m


def _upsample_matrix(hc, wc, h, w):
    hi = (np.arange(h) * hc) // h
    wi = (np.arange(w) * wc) // w
    src = (hi[:, None] * wc + wi[None, :]).reshape(-1)
    return jnp.asarray(
        (np.arange(hc * wc)[:, None] == src[None, :]).astype(np.float32),
        dtype=jnp.bfloat16)


def kernel(x1, x2, x3,
           output1_w, output1_scale, output1_bias,
           output2_w, output2_scale, output2_bias,
           output3_w, output3_scale, output3_bias,
           merge1_w, merge1_scale, merge1_bias,
           merge2_w, merge2_scale, merge2_bias):
    N, C1, H1, W1 = x1.shape
    _, C2, H2, W2 = x2.shape
    _, C3, H3, W3 = x3.shape
    HW1, HW2, HW3 = H1 * W1, H2 * W2, H3 * W3
    Cout = output1_w.shape[1]

    x1f = x1.reshape(N, C1, HW1)
    x2f = x2.reshape(N, C2, HW2)
    x3f = x3.reshape(N, C3, HW3)

    def pack1x1(w):
        return w.T.astype(jnp.bfloat16)

    def pack3x3(w):
        cout = w.shape[3]
        return (jnp.transpose(w, (3, 0, 1, 2)).reshape(cout, -1)
                .astype(jnp.bfloat16))

    w1 = pack1x1(output1_w)
    w2 = pack1x1(output2_w)
    w3 = pack1x1(output3_w)
    wm1 = pack3x3(merge1_w)
    wm2 = pack3x3(merge2_w)

    u21 = _upsample_matrix(H2, W2, H1, W1)
    u32 = _upsample_matrix(H3, W3, H2, W2)

    def col_masks(h, w):
        ww = np.arange(h * w) % w
        mneg = jnp.asarray((ww > 0).astype(np.float32).reshape(1, -1),
                           dtype=jnp.bfloat16)
        mpos = jnp.asarray((ww < w - 1).astype(np.float32).reshape(1, -1),
                           dtype=jnp.bfloat16)
        return mneg, mpos

    mneg1, mpos1 = col_masks(H1, W1)
    mneg2, mpos2 = col_masks(H2, W2)

    pad1 = max(128, W1 + 1)
    pad2 = max(128, W2 + 1)

    const = lambda *shape: pl.BlockSpec(shape, lambda n: tuple(0 for _ in shape))
    pern = lambda c, hw: pl.BlockSpec((1, c, hw), lambda n: (n, 0, 0))

    o1f, o2f, o3f = pl.pallas_call(
        functools.partial(_neck_kernel, H1=H1, W1=W1, H2=H2, W2=W2,
                          pad1=pad1, pad2=pad2),
        out_shape=[jax.ShapeDtypeStruct((N, Cout, HW1), jnp.float32),
                   jax.ShapeDtypeStruct((N, Cout, HW2), jnp.float32),
                   jax.ShapeDtypeStruct((N, Cout, HW3), jnp.float32)],
        grid=(N,),
        in_specs=[
            pern(C1, HW1), pern(C2, HW2), pern(C3, HW3),
            const(Cout, C1), const(Cout, 1), const(Cout, 1),
            const(Cout, C2), const(Cout, 1), const(Cout, 1),
            const(Cout, C3), const(Cout, 1), const(Cout, 1),
            const(Cout, 9 * Cout), const(Cout, 1), const(Cout, 1),
            const(Cout, 9 * Cout), const(Cout, 1), const(Cout, 1),
            const(HW2, HW1), const(HW3, HW2),
            const(1, HW1), const(1, HW1), const(1, HW2), const(1, HW2),
        ],
        out_specs=[pern(Cout, HW1), pern(Cout, HW2), pern(Cout, HW3)],
        scratch_shapes=[
            pltpu.VMEM((Cout, HW1 + 2 * pad1), jnp.bfloat16),
            pltpu.VMEM((Cout, HW2 + 2 * pad2), jnp.bfloat16),
        ],
        compiler_params=pltpu.CompilerParams(
            dimension_semantics=("parallel",),
            vmem_limit_bytes=100 * 1024 * 1024,
        ),
    )(x1f, x2f, x3f,
      w1, output1_scale.reshape(-1, 1), output1_bias.reshape(-1, 1),
      w2, output2_scale.reshape(-1, 1), output2_bias.reshape(-1, 1),
      w3, output3_scale.reshape(-1, 1), output3_bias.reshape(-1, 1),
      wm1, merge1_scale.reshape(-1, 1), merge1_bias.reshape(-1, 1),
      wm2, merge2_scale.reshape(-1, 1), merge2_bias.reshape(-1, 1),
      u21, u32, mneg1, mpos1, mneg2, mpos2)

    return [o1f.reshape(N, Cout, H1, W1),
            o2f.reshape(N, Cout, H2, W2),
            o3f.reshape(N, Cout, H3, W3)]
